# speedup vs baseline: 2.9426x; 2.9426x over previous
import functools

import jax
import jax.numpy as jnp
from jax import lax
from jax.experimental import pallas as pl
from jax.experimental.pallas import tpu as pltpu
from jax.experimental.pallas import tpu_sc as plsc

_DROPOUT_P = 0.5
_SCALE = 1.0 / (1.0 - _DROPOUT_P)
_MASK_BLK = 2000
_CH = 128
_NBUF = 4


def _mask_body(w_ref, k_ref, o_ref):
    o_ref[...] = jnp.where(k_ref[...], w_ref[...] * _SCALE, 0.0)


def _masked_table(weight_raw, keep):
    v, d = weight_raw.shape
    blk = _MASK_BLK
    return pl.pallas_call(
        _mask_body,
        grid=(v // blk,),
        in_specs=[
            pl.BlockSpec((blk, d), lambda i: (i, 0)),
            pl.BlockSpec((blk, d), lambda i: (i, 0)),
        ],
        out_specs=pl.BlockSpec((blk, d), lambda i: (i, 0)),
        out_shape=jax.ShapeDtypeStruct((v, d), jnp.float32),
    )(weight_raw, keep)


@functools.lru_cache(maxsize=None)
def _make_gather(v, d, b):
    info = plsc.get_sparse_core_info()
    nc, ns = info.num_cores, info.num_subcores
    nw = nc * ns
    per_w = b // nw
    nch = per_w // _CH
    assert per_w % _CH == 0 and nch % _NBUF == 0 and b == nw * per_w

    mesh = plsc.VectorSubcoreMesh(core_axis_name="c", subcore_axis_name="s")

    @functools.partial(
        pl.kernel,
        mesh=mesh,
        out_type=jax.ShapeDtypeStruct((b, d), jnp.float32),
        scratch_types=(
            [
                pltpu.VMEM((nch, _CH), jnp.int32),
                pltpu.VMEM((_NBUF, _CH, d), jnp.float32),
            ]
            + [pltpu.SemaphoreType.DMA] * (2 * _NBUF)
        ),
    )
    def gather(table_hbm, idx_hbm, out_hbm, idx_v, bufs, *sems):
        gsems, wsems = sems[:_NBUF], sems[_NBUF:]
        wid = lax.axis_index("s") * nc + lax.axis_index("c")
        r0 = wid * nch
        pltpu.sync_copy(idx_hbm.at[pl.ds(r0, nch)], idx_v)

        def g_start(j, slot):
            pltpu.async_copy(table_hbm.at[idx_v.at[j]], bufs.at[slot],
                             gsems[slot])

        def g_wait(slot):
            pltpu.make_async_copy(table_hbm.at[idx_v.at[0]], bufs.at[slot],
                                  gsems[slot]).wait()

        def w_start(j, slot):
            pltpu.async_copy(bufs.at[slot],
                             out_hbm.at[pl.ds((r0 + j) * _CH, _CH)],
                             wsems[slot])

        def w_wait(slot):
            pltpu.make_async_copy(bufs.at[slot], out_hbm.at[pl.ds(0, _CH)],
                                  wsems[slot]).wait()

        for slot in range(_NBUF):
            g_start(slot, slot)

        def outer(o, carry):
            j0 = o * _NBUF
            for slot in range(_NBUF):
                j = j0 + slot
                g_wait(slot)
                w_start(j, slot)
                nxt = j + _NBUF

                @pl.when(nxt < nch)
                def _():
                    w_wait(slot)
                    g_start(nxt, slot)

            return carry

        lax.fori_loop(0, nch // _NBUF, outer, 0)
        for slot in range(_NBUF):
            w_wait(slot)

    return gather


def kernel(input, weight_raw):
    batch, hist = input.shape
    v, d = weight_raw.shape
    b = batch * hist
    keep = jax.random.bernoulli(jax.random.key(42), 1.0 - _DROPOUT_P,
                                weight_raw.shape)
    w = _masked_table(weight_raw, keep)
    idx = input.reshape(b // _CH, _CH).astype(jnp.int32)
    out = _make_gather(v, d, b)(w, idx)
    return out.reshape(batch, hist, d)


# speedup vs baseline: 4.6484x; 1.5797x over previous
import functools

import jax
import jax.numpy as jnp
from jax import lax
from jax.experimental import pallas as pl
from jax.experimental.pallas import tpu as pltpu
from jax.experimental.pallas import tpu_sc as plsc

_DROPOUT_P = 0.5
_SCALE = 1.0 / (1.0 - _DROPOUT_P)
_MASK_BLK = 2000
_NBUF = 4


def _mask_body(w_ref, k_ref, o_ref):
    o_ref[...] = jnp.where(k_ref[...], w_ref[...] * _SCALE, 0.0)


def _masked_table(weight_raw, keep):
    v, d = weight_raw.shape
    blk = _MASK_BLK
    return pl.pallas_call(
        _mask_body,
        grid=(v // blk,),
        in_specs=[
            pl.BlockSpec((blk, d), lambda i: (i, 0)),
            pl.BlockSpec((blk, d), lambda i: (i, 0)),
        ],
        out_specs=pl.BlockSpec((blk, d), lambda i: (i, 0)),
        out_shape=jax.ShapeDtypeStruct((v, d), jnp.float32),
    )(weight_raw, keep)


@functools.lru_cache(maxsize=None)
def _make_gather(v, d, batch, hist):
    info = plsc.get_sparse_core_info()
    nc, ns = info.num_cores, info.num_subcores
    nw = nc * ns
    per_w = batch // nw
    assert batch == nw * per_w and per_w % _NBUF == 0

    mesh = plsc.VectorSubcoreMesh(core_axis_name="c", subcore_axis_name="s")

    @functools.partial(
        pl.kernel,
        mesh=mesh,
        out_type=jax.ShapeDtypeStruct((batch, hist, d), jnp.float32),
        scratch_types=(
            [
                pltpu.VMEM((per_w, hist), jnp.int32),
                pltpu.VMEM((_NBUF, hist, d), jnp.float32),
            ]
            + [pltpu.SemaphoreType.DMA] * (2 * _NBUF)
        ),
    )
    def gather(table_hbm, idx_hbm, out_hbm, idx_v, bufs, *sems):
        gsems, wsems = sems[:_NBUF], sems[_NBUF:]
        wid = lax.axis_index("s") * nc + lax.axis_index("c")
        b0 = wid * per_w
        pltpu.sync_copy(idx_hbm.at[pl.ds(b0, per_w)], idx_v)

        def g_start(i, slot):
            pltpu.async_copy(table_hbm.at[idx_v.at[i]], bufs.at[slot],
                             gsems[slot])

        def g_wait(slot):
            pltpu.make_async_copy(table_hbm.at[idx_v.at[0]], bufs.at[slot],
                                  gsems[slot]).wait()

        def w_start(i, slot):
            pltpu.async_copy(bufs.at[slot], out_hbm.at[b0 + i], wsems[slot])

        def w_wait(slot):
            pltpu.make_async_copy(bufs.at[slot], out_hbm.at[0],
                                  wsems[slot]).wait()

        for slot in range(_NBUF):
            g_start(slot, slot)

        def outer(o, carry):
            i0 = o * _NBUF
            for slot in range(_NBUF):
                i = i0 + slot
                g_wait(slot)
                w_start(i, slot)
                nxt = i + _NBUF

                @pl.when(nxt < per_w)
                def _():
                    w_wait(slot)
                    g_start(nxt, slot)

            return carry

        lax.fori_loop(0, per_w // _NBUF, outer, 0)
        for slot in range(_NBUF):
            w_wait(slot)

    return gather


def kernel(input, weight_raw):
    batch, hist = input.shape
    v, d = weight_raw.shape
    keep = jax.random.bernoulli(jax.random.key(42), 1.0 - _DROPOUT_P,
                                weight_raw.shape)
    w = _masked_table(weight_raw, keep)
    idx = input.astype(jnp.int32)
    return _make_gather(v, d, batch, hist)(w, idx)


# speedup vs baseline: 9.1474x; 1.9679x over previous
import functools

import jax
import jax.numpy as jnp
from jax import lax
from jax.experimental import pallas as pl
from jax.experimental.pallas import tpu as pltpu
from jax.experimental.pallas import tpu_sc as plsc

_SCALE = 2.0
_MASK_BLK = 2000
_CH = 128
_NBUF = 4
_NSPLIT = 2

_KS0 = 0
_KS1 = 42
_KS2 = 0x1BD11BDA ^ _KS0 ^ _KS1
_ROT_A = (13, 15, 26, 6)
_ROT_B = (17, 29, 16, 24)


def _threefry_signbit_keep(flat_s32):
    x1 = flat_s32.astype(jnp.uint32) + jnp.uint32(_KS1)
    x0 = jnp.zeros_like(x1)

    def rounds(x0, x1, rots):
        for r in rots:
            x0 = x0 + x1
            x1 = ((x1 << jnp.uint32(r)) | (x1 >> jnp.uint32(32 - r))) ^ x0
        return x0, x1

    x0, x1 = rounds(x0, x1, _ROT_A)
    x0 = x0 + jnp.uint32(_KS1)
    x1 = x1 + jnp.uint32(_KS2 + 1)
    x0, x1 = rounds(x0, x1, _ROT_B)
    x0 = x0 + jnp.uint32(_KS2)
    x1 = x1 + jnp.uint32(_KS0 + 2)
    x0, x1 = rounds(x0, x1, _ROT_A)
    x0 = x0 + jnp.uint32(_KS0)
    x1 = x1 + jnp.uint32(_KS1 + 3)
    x0, x1 = rounds(x0, x1, _ROT_B)
    x0 = x0 + jnp.uint32(_KS1)
    x1 = x1 + jnp.uint32(_KS2 + 4)
    x0, x1 = rounds(x0, x1, _ROT_A)
    x0 = x0 + jnp.uint32(_KS2)
    x1 = x1 + jnp.uint32(_KS0 + 5)
    return (x0 ^ x1) < jnp.uint32(0x80000000)


def _masked_packed_half(weight_raw, half, dc):
    v, d = weight_raw.shape
    v2 = v // 2
    blk = _MASK_BLK
    nblk = v2 // blk
    c0 = half * dc

    def body(wa_ref, wb_ref, o_ref):
        i = pl.program_id(0)
        pr = lax.broadcasted_iota(jnp.int32, (blk, 2 * dc), 0)
        q = lax.broadcasted_iota(jnp.int32, (blk, 2 * dc), 1)
        qh = q // dc
        flat = (i * blk + pr + qh * v2) * d + c0 + (q - qh * dc)
        keep = _threefry_signbit_keep(flat)
        wcomb = jnp.concatenate(
            [wa_ref[:, c0:c0 + dc], wb_ref[:, c0:c0 + dc]], axis=1)
        o_ref[...] = jnp.where(keep, wcomb * _SCALE, 0.0)

    return pl.pallas_call(
        body,
        grid=(nblk,),
        in_specs=[
            pl.BlockSpec((blk, d), lambda i: (i, 0)),
            pl.BlockSpec((blk, d), lambda i: (i + nblk, 0)),
        ],
        out_specs=pl.BlockSpec((blk, 2 * dc), lambda i: (i, 0)),
        out_shape=jax.ShapeDtypeStruct((v2, 2 * dc), jnp.float32),
    )(weight_raw, weight_raw)


@functools.lru_cache(maxsize=None)
def _make_gather_half(v, d, dc, batch, hist, half):
    info = plsc.get_sparse_core_info()
    nc, ns = info.num_cores, info.num_subcores
    nw = nc * ns
    per_w = batch // nw
    assert batch == nw * per_w and per_w == _NBUF * _CH

    mesh = plsc.VectorSubcoreMesh(core_axis_name="c", subcore_axis_name="s")

    @functools.partial(
        pl.kernel,
        mesh=mesh,
        out_type=(),
        scratch_types=(
            [
                pltpu.VMEM((hist * per_w,), jnp.int32),
                pltpu.VMEM((_NBUF, _CH, dc), jnp.float32),
            ]
            + [pltpu.SemaphoreType.DMA] * (2 * _NBUF)
        ),
        compiler_params=pltpu.CompilerParams(use_tc_tiling_on_sc=False),
    )
    def gather(table_hbm, idx_hbm, out_hbm, idx_v, bufs, *sems):
        gsems, wsems = sems[:_NBUF], sems[_NBUF:]
        wid = lax.axis_index("s") * nc + lax.axis_index("c")
        b0 = wid * per_w
        pltpu.sync_copy(idx_hbm.at[wid], idx_v)

        def g_start(h, slot):
            pltpu.async_copy(
                table_hbm.at[idx_v.at[pl.ds(h * per_w + slot * _CH, _CH)]],
                bufs.at[slot], gsems[slot])

        def g_wait(slot):
            pltpu.make_async_copy(table_hbm.at[idx_v.at[pl.ds(0, _CH)]],
                                  bufs.at[slot], gsems[slot]).wait()

        def w_start(h, slot):
            pltpu.async_copy(
                bufs.at[slot],
                out_hbm.at[h, pl.ds(b0 + slot * _CH, _CH),
                           pl.ds(half * dc, dc)],
                wsems[slot])

        def w_wait(slot):
            pltpu.make_async_copy(
                bufs.at[slot],
                out_hbm.at[0, pl.ds(0, _CH), pl.ds(half * dc, dc)],
                wsems[slot]).wait()

        for slot in range(_NBUF):
            g_start(0, slot)

        def outer(h, carry):
            for slot in range(_NBUF):
                g_wait(slot)
                w_start(h, slot)

                @pl.when(h + 1 < hist)
                def _():
                    w_wait(slot)
                    g_start(h + 1, slot)

            return carry

        lax.fori_loop(0, hist, outer, 0)
        for slot in range(_NBUF):
            w_wait(slot)

    return gather


def kernel(input, weight_raw):
    batch, hist = input.shape
    v, d = weight_raw.shape
    nw = 32
    per_w = batch // nw
    dc = d // _NSPLIT
    idx32 = input.astype(jnp.int32)
    idx32 = 2 * idx32 - jnp.where(idx32 >= v // 2, v - 1, 0)
    idx = (idx32.T.reshape(hist, nw, per_w).transpose(1, 0, 2)
           .reshape(nw, hist * per_w))
    out_ref = jax.new_ref(pl.empty((hist, batch, d), jnp.float32))
    for half in range(_NSPLIT):
        m_packed = _masked_packed_half(weight_raw, half, dc)
        table = m_packed.reshape(v, dc)
        _make_gather_half(v, d, dc, batch, hist, half)(table, idx, out_ref)
    out = out_ref[...]
    return out.transpose(1, 0, 2)
